# baseline (speedup 1.0000x reference)
import functools

import jax
import jax.numpy as jnp
from jax import lax
from jax.experimental import pallas as pl
from jax.experimental.pallas import tpu as pltpu
from jax.experimental.pallas import tpu_sc as plsc

B, T, P, E, V = 4096, 50, 8, 64, 1000
N = B * T
NC, NS = 2, 16
NW = NC * NS
TOK_W = N // NW
CHUNK = 400
NCH = TOK_W // CHUNK
L = 16
WPR = E // 2
JBLK = CHUNK // L


def _tree_sum(vals):
    while len(vals) > 1:
        vals = [vals[i] + vals[i + 1] for i in range(0, len(vals) - 1, 2)] + (
            [vals[-1]] if len(vals) % 2 else [])
    return vals[0]


def _body(ids_hbm, tbl_hbm, out_hbm, tbl_v, ids0, ids1, out0, out1,
          is0, is1, os0, os1):
    wid = lax.axis_index("s") * NC + lax.axis_index("c")
    ids_bufs = [ids0, ids1]
    out_bufs = [out0, out1]
    isems = [is0, is1]
    osems = [os0, os1]

    iota = lax.iota(jnp.int32, L)
    iota8 = iota * P
    iota64 = iota * E

    ids_base = wid * (TOK_W * P)
    out_base = wid * (TOK_W * E)

    def ids_copy(c, s):
        return pltpu.make_async_copy(
            ids_hbm.at[pl.ds(ids_base + c * (CHUNK * P), CHUNK * P)],
            ids_bufs[s], isems[s])

    def out_copy(c, s):
        return pltpu.make_async_copy(
            out_bufs[s],
            out_hbm.at[pl.ds(out_base + c * (CHUNK * E), CHUNK * E)],
            osems[s])

    ids_copy(0, 0).start()
    pltpu.sync_copy(tbl_hbm, tbl_v)
    ids_copy(1, 1).start()

    def chunk_iter(cc, _0):
        for s in (0, 1):
            c = cc * 2 + s
            ids_copy(c, s).wait()

            @pl.when(cc > 0)
            def _wait_out():
                out_copy(c - 2, s).wait()

            idsbuf = ids_bufs[s]
            outbuf = out_bufs[s]

            def blk_body(j, _1, idsbuf=idsbuf, outbuf=outbuf):
                ibase = j * (L * P)
                obase = j * (L * E)
                ids_p = [plsc.load_gather(idsbuf, [ibase + p + iota8])
                         for p in range(P)]
                cnt = _tree_sum([(ip != 0).astype(jnp.int32)
                                 for ip in ids_p])
                rcp = 1.0 / jnp.maximum(cnt.astype(jnp.float32), 1.0)
                rows = [ip * WPR for ip in ids_p]
                for w in range(WPR):
                    sm = _tree_sum([
                        plsc.bitcast(plsc.load_gather(tbl_v, [rows[p] + w]),
                                     jnp.bfloat16)
                        for p in range(P)
                    ])
                    a, b = plsc.unpack(sm,
                                       format=plsc.PackFormat.INTERLEAVED)
                    plsc.store_scatter(outbuf, [obase + w + iota64],
                                       a * rcp)
                    plsc.store_scatter(outbuf, [obase + w + WPR + iota64],
                                       b * rcp)
                return _1

            lax.fori_loop(0, JBLK, blk_body, None)
            out_copy(c, s).start()

            @pl.when(c + 2 < NCH)
            def _start_next():
                ids_copy(c + 2, s).start()
        return _0

    lax.fori_loop(0, NCH // 2, chunk_iter, None)
    for s in (0, 1):
        out_copy(NCH - 2 + s, s).wait()


@functools.partial(pl.kernel,
                   out_type=jax.ShapeDtypeStruct((N * E,), jnp.float32),
                   mesh=plsc.VectorSubcoreMesh(core_axis_name="c",
                                               subcore_axis_name="s"),
                   compiler_params=pltpu.CompilerParams(
                       needs_layout_passes=False,
                       use_tc_tiling_on_sc=False),
                   scratch_types=[
                       pltpu.VMEM((V * WPR,), jnp.int32),
                       pltpu.VMEM((CHUNK * P,), jnp.int32),
                       pltpu.VMEM((CHUNK * P,), jnp.int32),
                       pltpu.VMEM((CHUNK * E,), jnp.float32),
                       pltpu.VMEM((CHUNK * E,), jnp.float32),
                       pltpu.SemaphoreType.DMA,
                       pltpu.SemaphoreType.DMA,
                       pltpu.SemaphoreType.DMA,
                       pltpu.SemaphoreType.DMA,
                   ])
def _pooled_embed(ids_hbm, tbl_hbm, out_hbm, *scratch):
    _body(ids_hbm, tbl_hbm, out_hbm, *scratch)


def kernel(phone_ids, embed_table):
    tb = embed_table.astype(jnp.bfloat16)
    packed = lax.bitcast_convert_type(
        jnp.stack([tb[:, :32], tb[:, 32:]], axis=-1), jnp.int32)
    out = _pooled_embed(phone_ids.reshape(-1), packed.reshape(-1))
    return out.reshape(B, T, E)


# speedup vs baseline: 3.4948x; 3.4948x over previous
import functools

import jax
import jax.numpy as jnp
from jax import lax
from jax.experimental import pallas as pl
from jax.experimental.pallas import tpu as pltpu
from jax.experimental.pallas import tpu_sc as plsc

B, T, P, E, V = 4096, 50, 8, 64, 1000
N = B * T
NC, NS = 2, 16
NW = NC * NS
TOK_W = N // NW
CHUNK = 400
CB = CHUNK // T
NCH = TOK_W // CHUNK
L = 16
WPR = E // 2


def _tree_sum(vals):
    while len(vals) > 1:
        vals = [vals[i] + vals[i + 1] for i in range(0, len(vals) - 1, 2)] + (
            [vals[-1]] if len(vals) % 2 else [])
    return vals[0]


def _body(ids_hbm, tbl_hbm, out_hbm, tbl_v, ids0, ids1, out0, out1,
          is0, is1, os0, os1):
    wid = lax.axis_index("s") * NC + lax.axis_index("c")
    ids_bufs = [ids0, ids1]
    out_bufs = [out0, out1]
    isems = [is0, is1]
    osems = [os0, os1]

    iota = lax.iota(jnp.int32, L)
    offs = [g * L + iota for g in range(2)]
    splat_idx = [jnp.full((L, 1), k, jnp.int32) for k in range(L)]
    gdn = lax.GatherDimensionNumbers(offset_dims=(),
                                     collapsed_slice_dims=(0,),
                                     start_index_map=(0,))

    def splat(vec, k):
        return lax.gather(vec, splat_idx[k], gdn, (1,),
                          mode=lax.GatherScatterMode.PROMISE_IN_BOUNDS)

    ids_base = wid * (TOK_W * P)
    out_row = wid * (TOK_W // T)

    def start_ids(c):
        return pltpu.async_copy(
            ids_hbm.at[pl.ds(ids_base + c * (CHUNK * P), CHUNK * P)],
            ids_bufs[c % 2], isems[c % 2])

    h_ids = start_ids(0)
    pltpu.sync_copy(tbl_hbm, tbl_v)

    h_out = [None, None]
    for c in range(NCH):
        h_ids.wait()
        if c + 1 < NCH:
            h_ids = start_ids(c + 1)
        if h_out[c % 2] is not None:
            h_out[c % 2].wait()

        idsbuf = ids_bufs[c % 2]
        outbuf = out_bufs[c % 2]

        def pair_body(j, _, idsbuf=idsbuf, outbuf=outbuf):
            idsv = idsbuf[pl.ds(j * L, L)]
            rows = idsv * WPR
            m = (idsv != 0).astype(jnp.int32)
            cum = plsc.cumsum(m)
            c0 = splat(cum, 7)
            c1 = splat(cum, 15) - c0
            r0 = 1.0 / jnp.maximum(c0.astype(jnp.float32), 1.0)
            r1 = 1.0 / jnp.maximum(c1.astype(jnp.float32), 1.0)
            br = j // (T // 2)
            brv = jnp.full((L,), br, jnp.int32)
            for t in range(2):
                rr = r0 if t == 0 else r1
                tok = (j % (T // 2)) * 2 + t
                tokv = jnp.full((L,), tok, jnp.int32)
                sps = [splat(rows, t * 8 + p) for p in range(8)]
                for g in range(2):
                    vals = [
                        plsc.bitcast(
                            plsc.load_gather(tbl_v, [sps[p] + offs[g]]),
                            jnp.bfloat16)
                        for p in range(8)
                    ]
                    s = _tree_sum(vals)
                    a, b = plsc.unpack(s, format=plsc.PackFormat.INTERLEAVED)
                    plsc.store_scatter(outbuf, [brv, tokv, offs[g]], a * rr)
                    plsc.store_scatter(outbuf, [brv, tokv, 32 + offs[g]],
                                       b * rr)
            return _

        lax.fori_loop(0, CHUNK // 2, pair_body, None, unroll=2)

        h_out[c % 2] = pltpu.async_copy(
            outbuf,
            out_hbm.at[pl.ds(out_row + c * CB, CB)],
            osems[c % 2])

    h_out[(NCH - 2) % 2].wait()
    h_out[(NCH - 1) % 2].wait()


@functools.partial(pl.kernel,
                   out_type=jax.ShapeDtypeStruct((B, T, E), jnp.float32),
                   mesh=plsc.VectorSubcoreMesh(core_axis_name="c",
                                               subcore_axis_name="s"),
                   compiler_params=pltpu.CompilerParams(
                       needs_layout_passes=False,
                       use_tc_tiling_on_sc=False),
                   scratch_types=[
                       pltpu.VMEM((V * WPR,), jnp.int32),
                       pltpu.VMEM((CHUNK * P,), jnp.int32),
                       pltpu.VMEM((CHUNK * P,), jnp.int32),
                       pltpu.VMEM((CB, T, E), jnp.float32),
                       pltpu.VMEM((CB, T, E), jnp.float32),
                       pltpu.SemaphoreType.DMA,
                       pltpu.SemaphoreType.DMA,
                       pltpu.SemaphoreType.DMA,
                       pltpu.SemaphoreType.DMA,
                   ])
def _pooled_embed(ids_hbm, tbl_hbm, out_hbm, *scratch):
    _body(ids_hbm, tbl_hbm, out_hbm, *scratch)


def kernel(phone_ids, embed_table):
    tb = embed_table.astype(jnp.bfloat16)
    packed = lax.bitcast_convert_type(
        jnp.stack([tb[:, :32], tb[:, 32:]], axis=-1), jnp.int32)
    return _pooled_embed(phone_ids.reshape(-1), packed.reshape(-1))
